# Initial kernel scaffold; baseline (speedup 1.0000x reference)
import jax
import jax.numpy as jnp
from jax.experimental import pallas as pl


def kernel(t, x, edge_index, x0, W, att_src, att_dst, bias, alpha_train, beta_train):
    raise NotImplementedError("write your pallas kernel here")


# speedup vs baseline: 2.0520x; 2.0520x over previous
import functools
import jax
import jax.numpy as jnp
from jax.experimental import pallas as pl
from jax.experimental.pallas import tpu as pltpu

_SLOPE = 0.2


def _blk(n, cap):
    b = min(n, cap)
    while n % b:
        b -= 1
    return b


def _dense_kernel(x_ref, w_ref, a_ref, h_bf_ref, a_out_ref):
    h = jnp.dot(x_ref[...], w_ref[...], preferred_element_type=jnp.float32)
    h_bf_ref[...] = h.astype(jnp.bfloat16)
    a_out_ref[...] = jnp.dot(h, a_ref[...], preferred_element_type=jnp.float32)


def _gather_kernel(nj, heads, dim, src_ref, dst_ref, h_ref, a_ref,
                   m_ref, w_ref, ae_scr):
    j = pl.program_id(1)

    @pl.when(j == 0)
    def _():
        m_ref[...] = jnp.zeros_like(m_ref)
        ae_scr[...] = jnp.zeros_like(ae_scr)

    src = src_ref[0, 0, :]
    dst = dst_ref[0, 0, :]
    eb = src.shape[0]
    nb = h_ref.shape[0]
    cols = jax.lax.broadcasted_iota(jnp.int32, (eb, nb), 1) + j * nb
    p_src = jnp.where(cols == src[:, None], 1.0, 0.0)
    p_dst = jnp.where(cols == dst[:, None], 1.0, 0.0)
    m_ref[...] += jnp.dot(p_src.astype(jnp.bfloat16), h_ref[...],
                          preferred_element_type=jnp.float32).astype(jnp.bfloat16)
    a_blk = a_ref[...]
    ga = jnp.dot(p_src, a_blk[:, :heads], preferred_element_type=jnp.float32)
    gb = jnp.dot(p_dst, a_blk[:, heads:], preferred_element_type=jnp.float32)
    ae_scr[...] += jnp.concatenate([ga, gb], axis=1)

    @pl.when(j == nj - 1)
    def _():
        ae = ae_scr[...]
        e = ae[:, :heads] + ae[:, heads:]
        e = jnp.where(e > 0, e, _SLOPE * e)
        wb = jnp.exp(e).astype(jnp.bfloat16)
        w_ref[...] = jnp.concatenate(
            [wb.astype(jnp.float32), jnp.zeros_like(ae[:, :heads])], axis=1)
        m = m_ref[...]
        m_ref[...] = jnp.concatenate(
            [m[:, h * dim:(h + 1) * dim] * wb[:, h:h + 1] for h in range(heads)],
            axis=1)


def _scatter_kernel(ni, heads, dim, dst_ref, m_ref, w_ref, x_ref, x0_ref,
                    bias_ref, par_ref, f_ref, acc_scr, den_scr):
    n = pl.program_id(0)
    i = pl.program_id(1)

    @pl.when(i == 0)
    def _():
        acc_scr[...] = jnp.zeros_like(acc_scr)
        den_scr[...] = jnp.zeros_like(den_scr)

    dst = dst_ref[0, 0, :]
    eb = dst.shape[0]
    nb = acc_scr.shape[0]
    rows = jax.lax.broadcasted_iota(jnp.int32, (nb, eb), 0) + n * nb
    pt = jnp.where(rows == dst[None, :], 1.0, 0.0)
    acc_scr[...] += jnp.dot(pt.astype(jnp.bfloat16), m_ref[...],
                            preferred_element_type=jnp.float32)
    den_scr[...] += jnp.dot(pt, w_ref[...][:, :heads],
                            preferred_element_type=jnp.float32)

    @pl.when(i == ni - 1)
    def _():
        acc = acc_scr[...]
        den = den_scr[...]
        ax = acc[:, :dim] / (den[:, 0:1] + 1e-16)
        for h in range(1, heads):
            ax = ax + acc[:, h * dim:(h + 1) * dim] / (den[:, h:h + 1] + 1e-16)
        ax = ax * (1.0 / heads) + bias_ref[...]
        alpha = jax.nn.sigmoid(par_ref[0:1, 0:1])
        beta = par_ref[0:1, 1:2]
        f_ref[...] = alpha * (ax - x_ref[...]) + beta * x0_ref[...]


def kernel(t, x, edge_index, x0, W, att_src, att_dst, bias, alpha_train,
           beta_train):
    del t
    N, D = x.shape
    H = att_src.shape[0]
    E = edge_index.shape[1]
    HD = H * D
    NB = _blk(N, 1024)
    EB = _blk(E, 1024)
    nj = N // NB
    ni = E // EB

    A = jnp.zeros((HD, 2 * H), jnp.float32)
    for h in range(H):
        A = A.at[h * D:(h + 1) * D, h].set(att_src[h])
        A = A.at[h * D:(h + 1) * D, H + h].set(att_dst[h])

    h_bf, a = pl.pallas_call(
        _dense_kernel,
        grid=(nj,),
        in_specs=[
            pl.BlockSpec((NB, D), lambda i: (i, 0)),
            pl.BlockSpec((D, HD), lambda i: (0, 0)),
            pl.BlockSpec((HD, 2 * H), lambda i: (0, 0)),
        ],
        out_specs=[
            pl.BlockSpec((NB, HD), lambda i: (i, 0)),
            pl.BlockSpec((NB, 2 * H), lambda i: (i, 0)),
        ],
        out_shape=[
            jax.ShapeDtypeStruct((N, HD), jnp.bfloat16),
            jax.ShapeDtypeStruct((N, 2 * H), jnp.float32),
        ],
    )(x, W, A)

    src3 = edge_index[0].reshape(ni, 1, EB)
    dst3 = edge_index[1].reshape(ni, 1, EB)

    m, w = pl.pallas_call(
        functools.partial(_gather_kernel, nj, H, D),
        grid=(ni, nj),
        in_specs=[
            pl.BlockSpec((1, 1, EB), lambda i, j: (i, 0, 0)),
            pl.BlockSpec((1, 1, EB), lambda i, j: (i, 0, 0)),
            pl.BlockSpec((NB, HD), lambda i, j: (j, 0)),
            pl.BlockSpec((NB, 2 * H), lambda i, j: (j, 0)),
        ],
        out_specs=[
            pl.BlockSpec((EB, HD), lambda i, j: (i, 0)),
            pl.BlockSpec((EB, 2 * H), lambda i, j: (i, 0)),
        ],
        out_shape=[
            jax.ShapeDtypeStruct((E, HD), jnp.bfloat16),
            jax.ShapeDtypeStruct((E, 2 * H), jnp.float32),
        ],
        scratch_shapes=[pltpu.VMEM((EB, 2 * H), jnp.float32)],
    )(src3, dst3, h_bf, a)

    par = jnp.zeros((1, 128), jnp.float32)
    par = par.at[0, 0].set(alpha_train)
    par = par.at[0, 1].set(beta_train)

    f = pl.pallas_call(
        functools.partial(_scatter_kernel, ni, H, D),
        grid=(nj, ni),
        in_specs=[
            pl.BlockSpec((1, 1, EB), lambda n, i: (i, 0, 0)),
            pl.BlockSpec((EB, HD), lambda n, i: (i, 0)),
            pl.BlockSpec((EB, 2 * H), lambda n, i: (i, 0)),
            pl.BlockSpec((NB, D), lambda n, i: (n, 0)),
            pl.BlockSpec((NB, D), lambda n, i: (n, 0)),
            pl.BlockSpec((1, D), lambda n, i: (0, 0)),
            pl.BlockSpec((1, 128), lambda n, i: (0, 0)),
        ],
        out_specs=pl.BlockSpec((NB, D), lambda n, i: (n, 0)),
        out_shape=jax.ShapeDtypeStruct((N, D), jnp.float32),
        scratch_shapes=[
            pltpu.VMEM((NB, HD), jnp.float32),
            pltpu.VMEM((NB, H), jnp.float32),
        ],
    )(dst3, m, w, x, x0, bias.reshape(1, D), par)

    return f


# speedup vs baseline: 2.6850x; 1.3085x over previous
import functools
import jax
import jax.numpy as jnp
from jax.experimental import pallas as pl
from jax.experimental.pallas import tpu as pltpu

_SLOPE = 0.2


def _blk(n, cap):
    b = min(n, cap)
    while n % b:
        b -= 1
    return b


def _dense_kernel(x_ref, w_ref, a_ref, a_out_ref):
    h = jnp.dot(x_ref[...], w_ref[...], preferred_element_type=jnp.float32)
    a_out_ref[...] = jnp.dot(h, a_ref[...], preferred_element_type=jnp.float32)


def _gather_kernel(nj, heads, dim, src_ref, dst_ref, x_ref, a_ref, w_in_ref,
                   m_ref, w_ref, xg_scr, ae_scr):
    j = pl.program_id(1)

    @pl.when(j == 0)
    def _():
        xg_scr[...] = jnp.zeros_like(xg_scr)
        ae_scr[...] = jnp.zeros_like(ae_scr)

    src = src_ref[0, 0, :]
    dst = dst_ref[0, 0, :]
    eb = src.shape[0]
    nb = x_ref.shape[0]
    cols = jax.lax.broadcasted_iota(jnp.int32, (eb, nb), 1) + j * nb
    p_src = jnp.where(cols == src[:, None], 1.0, 0.0)
    p_dst = jnp.where(cols == dst[:, None], 1.0, 0.0)
    xg_scr[...] += jnp.dot(p_src.astype(jnp.bfloat16), x_ref[...],
                           preferred_element_type=jnp.float32)
    a_blk = a_ref[...]
    ga = jnp.dot(p_src, a_blk[:, :heads], preferred_element_type=jnp.float32)
    gb = jnp.dot(p_dst, a_blk[:, heads:], preferred_element_type=jnp.float32)
    ae_scr[...] += jnp.concatenate([ga, gb], axis=1)

    @pl.when(j == nj - 1)
    def _():
        ae = ae_scr[...]
        e = ae[:, :heads] + ae[:, heads:]
        e = jnp.where(e > 0, e, _SLOPE * e)
        wf = jnp.exp(e)
        w_ref[...] = jnp.concatenate([wf, jnp.zeros_like(wf)], axis=1)
        m = jnp.dot(xg_scr[...], w_in_ref[...],
                    preferred_element_type=jnp.float32)
        m_ref[...] = jnp.concatenate(
            [m[:, h * dim:(h + 1) * dim] * wf[:, h:h + 1] for h in range(heads)],
            axis=1).astype(jnp.bfloat16)


def _scatter_kernel(ni, heads, dim, dst_ref, m_ref, w_ref, x_ref, x0_ref,
                    bias_ref, par_ref, f_ref, acc_scr, den_scr):
    n = pl.program_id(0)
    i = pl.program_id(1)

    @pl.when(i == 0)
    def _():
        acc_scr[...] = jnp.zeros_like(acc_scr)
        den_scr[...] = jnp.zeros_like(den_scr)

    dst = dst_ref[0, 0, :]
    eb = dst.shape[0]
    nb = acc_scr.shape[0]
    rows = jax.lax.broadcasted_iota(jnp.int32, (nb, eb), 0) + n * nb
    pt = jnp.where(rows == dst[None, :], 1.0, 0.0)
    acc_scr[...] += jnp.dot(pt.astype(jnp.bfloat16), m_ref[...],
                            preferred_element_type=jnp.float32)
    den_scr[...] += jnp.dot(pt, w_ref[...][:, :heads],
                            preferred_element_type=jnp.float32)

    @pl.when(i == ni - 1)
    def _():
        acc = acc_scr[...]
        den = den_scr[...]
        ax = acc[:, :dim] / (den[:, 0:1] + 1e-16)
        for h in range(1, heads):
            ax = ax + acc[:, h * dim:(h + 1) * dim] / (den[:, h:h + 1] + 1e-16)
        ax = ax * (1.0 / heads) + bias_ref[...]
        alpha = jax.nn.sigmoid(par_ref[0:1, 0:1])
        beta = par_ref[0:1, 1:2]
        f_ref[...] = alpha * (ax - x_ref[...]) + beta * x0_ref[...]


def kernel(t, x, edge_index, x0, W, att_src, att_dst, bias, alpha_train,
           beta_train):
    del t
    N, D = x.shape
    H = att_src.shape[0]
    E = edge_index.shape[1]
    HD = H * D
    NB = _blk(N, 1024)
    EB = _blk(E, 1024)
    nj = N // NB
    ni = E // EB

    A = jnp.zeros((HD, 2 * H), jnp.float32)
    for h in range(H):
        A = A.at[h * D:(h + 1) * D, h].set(att_src[h])
        A = A.at[h * D:(h + 1) * D, H + h].set(att_dst[h])

    a = pl.pallas_call(
        _dense_kernel,
        grid=(nj,),
        in_specs=[
            pl.BlockSpec((NB, D), lambda i: (i, 0)),
            pl.BlockSpec((D, HD), lambda i: (0, 0)),
            pl.BlockSpec((HD, 2 * H), lambda i: (0, 0)),
        ],
        out_specs=pl.BlockSpec((NB, 2 * H), lambda i: (i, 0)),
        out_shape=jax.ShapeDtypeStruct((N, 2 * H), jnp.float32),
    )(x, W, A)

    src3 = edge_index[0].reshape(ni, 1, EB)
    dst3 = edge_index[1].reshape(ni, 1, EB)
    x_bf = x.astype(jnp.bfloat16)

    m, w = pl.pallas_call(
        functools.partial(_gather_kernel, nj, H, D),
        grid=(ni, nj),
        in_specs=[
            pl.BlockSpec((1, 1, EB), lambda i, j: (i, 0, 0)),
            pl.BlockSpec((1, 1, EB), lambda i, j: (i, 0, 0)),
            pl.BlockSpec((NB, D), lambda i, j: (j, 0)),
            pl.BlockSpec((NB, 2 * H), lambda i, j: (j, 0)),
            pl.BlockSpec((D, HD), lambda i, j: (0, 0)),
        ],
        out_specs=[
            pl.BlockSpec((EB, HD), lambda i, j: (i, 0)),
            pl.BlockSpec((EB, 2 * H), lambda i, j: (i, 0)),
        ],
        out_shape=[
            jax.ShapeDtypeStruct((E, HD), jnp.bfloat16),
            jax.ShapeDtypeStruct((E, 2 * H), jnp.float32),
        ],
        scratch_shapes=[
            pltpu.VMEM((EB, D), jnp.float32),
            pltpu.VMEM((EB, 2 * H), jnp.float32),
        ],
    )(src3, dst3, x_bf, a, W)

    par = jnp.zeros((1, 128), jnp.float32)
    par = par.at[0, 0].set(alpha_train)
    par = par.at[0, 1].set(beta_train)

    f = pl.pallas_call(
        functools.partial(_scatter_kernel, ni, H, D),
        grid=(nj, ni),
        in_specs=[
            pl.BlockSpec((1, 1, EB), lambda n, i: (i, 0, 0)),
            pl.BlockSpec((EB, HD), lambda n, i: (i, 0)),
            pl.BlockSpec((EB, 2 * H), lambda n, i: (i, 0)),
            pl.BlockSpec((NB, D), lambda n, i: (n, 0)),
            pl.BlockSpec((NB, D), lambda n, i: (n, 0)),
            pl.BlockSpec((1, D), lambda n, i: (0, 0)),
            pl.BlockSpec((1, 128), lambda n, i: (0, 0)),
        ],
        out_specs=pl.BlockSpec((NB, D), lambda n, i: (n, 0)),
        out_shape=jax.ShapeDtypeStruct((N, D), jnp.float32),
        scratch_shapes=[
            pltpu.VMEM((NB, HD), jnp.float32),
            pltpu.VMEM((NB, H), jnp.float32),
        ],
    )(dst3, m, w, x, x0, bias.reshape(1, D), par)

    return f


# speedup vs baseline: 2.6861x; 1.0004x over previous
import functools
import jax
import jax.numpy as jnp
from jax.experimental import pallas as pl
from jax.experimental.pallas import tpu as pltpu

_SLOPE = 0.2


def _blk(n, cap):
    b = min(n, cap)
    while n % b:
        b -= 1
    return b


def _dense_kernel(x_ref, w_ref, a_ref, a_out_ref):
    h = jnp.dot(x_ref[...], w_ref[...], preferred_element_type=jnp.float32)
    a_out_ref[...] = jnp.dot(h, a_ref[...], preferred_element_type=jnp.float32)


def _gather_kernel(nj, heads, dim, src_ref, dst_ref, x_ref, a_ref, w_in_ref,
                   m_ref, w_ref, xg_scr, ae_scr):
    j = pl.program_id(1)

    @pl.when(j == 0)
    def _():
        xg_scr[...] = jnp.zeros_like(xg_scr)
        ae_scr[...] = jnp.zeros_like(ae_scr)

    src = src_ref[0, 0, :]
    dst = dst_ref[0, 0, :]
    eb = src.shape[0]
    nb = x_ref.shape[0]
    cols = jax.lax.broadcasted_iota(jnp.int32, (eb, nb), 1) + j * nb
    p_src = jnp.where(cols == src[:, None], 1.0, 0.0)
    p_dst = jnp.where(cols == dst[:, None], 1.0, 0.0)
    xg_scr[...] += jnp.dot(p_src.astype(jnp.bfloat16), x_ref[...],
                           preferred_element_type=jnp.float32)
    a_blk = a_ref[...]
    ga = jnp.dot(p_src, a_blk[:, :heads], preferred_element_type=jnp.float32)
    gb = jnp.dot(p_dst, a_blk[:, heads:], preferred_element_type=jnp.float32)
    ae_scr[...] += jnp.concatenate([ga, gb], axis=1)

    @pl.when(j == nj - 1)
    def _():
        ae = ae_scr[...]
        e = ae[:, :heads] + ae[:, heads:]
        e = jnp.where(e > 0, e, _SLOPE * e)
        wf = jnp.exp(e)
        w_ref[...] = jnp.concatenate([wf, jnp.zeros_like(wf)], axis=1)
        m = jnp.dot(xg_scr[...], w_in_ref[...],
                    preferred_element_type=jnp.float32)
        m_ref[...] = jnp.concatenate(
            [m[:, h * dim:(h + 1) * dim] * wf[:, h:h + 1] for h in range(heads)],
            axis=1).astype(jnp.bfloat16)


def _scatter_kernel(ni, heads, dim, dst_ref, m_ref, w_ref, x_ref, x0_ref,
                    bias_ref, par_ref, f_ref, acc_scr, den_scr):
    n = pl.program_id(0)
    i = pl.program_id(1)

    @pl.when(i == 0)
    def _():
        acc_scr[...] = jnp.zeros_like(acc_scr)
        den_scr[...] = jnp.zeros_like(den_scr)

    dst = dst_ref[0, 0, :]
    eb = dst.shape[0]
    nb = acc_scr.shape[0]
    rows = jax.lax.broadcasted_iota(jnp.int32, (nb, eb), 0) + n * nb
    pt = jnp.where(rows == dst[None, :], 1.0, 0.0)
    acc_scr[...] += jnp.dot(pt.astype(jnp.bfloat16), m_ref[...],
                            preferred_element_type=jnp.float32)
    den_scr[...] += jnp.dot(pt, w_ref[...][:, :heads],
                            preferred_element_type=jnp.float32)

    @pl.when(i == ni - 1)
    def _():
        acc = acc_scr[...]
        den = den_scr[...]
        ax = acc[:, :dim] / (den[:, 0:1] + 1e-16)
        for h in range(1, heads):
            ax = ax + acc[:, h * dim:(h + 1) * dim] / (den[:, h:h + 1] + 1e-16)
        ax = ax * (1.0 / heads) + bias_ref[...]
        alpha = jax.nn.sigmoid(par_ref[0:1, 0:1])
        beta = par_ref[0:1, 1:2]
        f_ref[...] = alpha * (ax - x_ref[...]) + beta * x0_ref[...]


def kernel(t, x, edge_index, x0, W, att_src, att_dst, bias, alpha_train,
           beta_train):
    del t
    N, D = x.shape
    H = att_src.shape[0]
    E = edge_index.shape[1]
    HD = H * D
    NB = _blk(N, 1024)
    EB = _blk(E, 1024)
    nj = N // NB
    ni = E // EB

    A = jnp.zeros((HD, 2 * H), jnp.float32)
    for h in range(H):
        A = A.at[h * D:(h + 1) * D, h].set(att_src[h])
        A = A.at[h * D:(h + 1) * D, H + h].set(att_dst[h])

    a = pl.pallas_call(
        _dense_kernel,
        grid=(nj,),
        in_specs=[
            pl.BlockSpec((NB, D), lambda i: (i, 0)),
            pl.BlockSpec((D, HD), lambda i: (0, 0)),
            pl.BlockSpec((HD, 2 * H), lambda i: (0, 0)),
        ],
        out_specs=pl.BlockSpec((NB, 2 * H), lambda i: (i, 0)),
        out_shape=jax.ShapeDtypeStruct((N, 2 * H), jnp.float32),
        compiler_params=pltpu.CompilerParams(
            dimension_semantics=("parallel",)),
    )(x, W, A)

    src3 = edge_index[0].reshape(ni, 1, EB)
    dst3 = edge_index[1].reshape(ni, 1, EB)
    x_bf = x.astype(jnp.bfloat16)

    m, w = pl.pallas_call(
        functools.partial(_gather_kernel, nj, H, D),
        grid=(ni, nj),
        in_specs=[
            pl.BlockSpec((1, 1, EB), lambda i, j: (i, 0, 0)),
            pl.BlockSpec((1, 1, EB), lambda i, j: (i, 0, 0)),
            pl.BlockSpec((NB, D), lambda i, j: (j, 0)),
            pl.BlockSpec((NB, 2 * H), lambda i, j: (j, 0)),
            pl.BlockSpec((D, HD), lambda i, j: (0, 0)),
        ],
        out_specs=[
            pl.BlockSpec((EB, HD), lambda i, j: (i, 0)),
            pl.BlockSpec((EB, 2 * H), lambda i, j: (i, 0)),
        ],
        out_shape=[
            jax.ShapeDtypeStruct((E, HD), jnp.bfloat16),
            jax.ShapeDtypeStruct((E, 2 * H), jnp.float32),
        ],
        scratch_shapes=[
            pltpu.VMEM((EB, D), jnp.float32),
            pltpu.VMEM((EB, 2 * H), jnp.float32),
        ],
        compiler_params=pltpu.CompilerParams(
            dimension_semantics=("parallel", "arbitrary")),
    )(src3, dst3, x_bf, a, W)

    par = jnp.zeros((1, 128), jnp.float32)
    par = par.at[0, 0].set(alpha_train)
    par = par.at[0, 1].set(beta_train)

    f = pl.pallas_call(
        functools.partial(_scatter_kernel, ni, H, D),
        grid=(nj, ni),
        in_specs=[
            pl.BlockSpec((1, 1, EB), lambda n, i: (i, 0, 0)),
            pl.BlockSpec((EB, HD), lambda n, i: (i, 0)),
            pl.BlockSpec((EB, 2 * H), lambda n, i: (i, 0)),
            pl.BlockSpec((NB, D), lambda n, i: (n, 0)),
            pl.BlockSpec((NB, D), lambda n, i: (n, 0)),
            pl.BlockSpec((1, D), lambda n, i: (0, 0)),
            pl.BlockSpec((1, 128), lambda n, i: (0, 0)),
        ],
        out_specs=pl.BlockSpec((NB, D), lambda n, i: (n, 0)),
        out_shape=jax.ShapeDtypeStruct((N, D), jnp.float32),
        scratch_shapes=[
            pltpu.VMEM((NB, HD), jnp.float32),
            pltpu.VMEM((NB, H), jnp.float32),
        ],
        compiler_params=pltpu.CompilerParams(
            dimension_semantics=("parallel", "arbitrary")),
    )(dst3, m, w, x, x0, bias.reshape(1, D), par)

    return f
